# baseline (speedup 1.0000x reference)
import functools

import jax
import jax.numpy as jnp
from jax import lax
from jax.experimental import pallas as pl
from jax.experimental.pallas import tpu as pltpu
from jax.experimental.pallas import tpu_sc as plsc

EMB = 64
SEQ = 200
SEQ_PAD = 208
NCHUNK = 13
BATCH = 4096
NWORKERS = 32
ROWS_PER_W = BATCH // NWORKERS
GROUP = 4
NGROUPS = ROWS_PER_W // GROUP
GSEQ = GROUP * SEQ
GCHUNK = GSEQ // 16
GA = 112
GB = 112
TAB_ROWS = GROUP * SEQ + 24


def _sc_body(seq_hbm, table_hbm, out_hbm,
             seq_g0, seq_g1, ia, ib, rows_v, tab_v, flags,
             sin0, sin1, sg, sout0, sout1):
    cid = lax.axis_index("c")
    sid = lax.axis_index("s")
    wid = sid * 2 + cid
    base = wid * ROWS_PER_W

    zeros16 = jnp.zeros((16,), jnp.int32)
    ones16 = jnp.ones((16,), jnp.int32)
    lane = lax.iota(jnp.int32, 16)
    tail_valid = lane < jnp.full((16,), 8, jnp.int32)
    scan_idx = [jnp.maximum(lane - (1 << k), zeros16) for k in range(4)]
    scan_msk = [lane >= jnp.full((16,), 1 << k, jnp.int32) for k in range(4)]
    bfly_idx = [lane ^ jnp.full((16,), 1 << k, jnp.int32) for k in range(4)]
    idx_last = jnp.full((16,), 15, jnp.int32)

    dnums = lax.GatherDimensionNumbers(
        offset_dims=(), collapsed_slice_dims=(0,), start_index_map=(0,))

    def _lanegather(x, idx):
        return lax.gather(x, idx[:, None], dnums, slice_sizes=(1,),
                          mode=lax.GatherScatterMode.PROMISE_IN_BOUNDS)

    def _cumsum16(m):
        s = m
        for k in range(4):
            g = _lanegather(s, scan_idx[k])
            s = s + jnp.where(scan_msk[k], g, zeros16)
        return s

    def _allsum16(x):
        s = x
        for k in range(4):
            s = s + _lanegather(s, bfly_idx[k])
        return s

    def gather_cps(rows_ref, row_off, sem):
        return (pltpu.make_async_copy(
                    table_hbm.at[ia],
                    rows_ref.at[pl.ds(row_off, GA)], sem),
                pltpu.make_async_copy(
                    table_hbm.at[ib],
                    rows_ref.at[pl.ds(row_off + GA, GB)], sem))

    def in_cp(g, seq_ref, sem):
        return pltpu.make_async_copy(
            seq_hbm.at[pl.ds((base + GROUP * g) * SEQ, GSEQ)],
            seq_ref.at[pl.ds(0, GSEQ)], sem)

    def out_fast_cp(g, sem):
        return pltpu.make_async_copy(
            tab_v.at[pl.ds(0, GSEQ)],
            out_hbm.at[pl.ds((base + GROUP * g) * SEQ, GSEQ)], sem)

    def out_slow_cp(g, sem):
        return pltpu.make_async_copy(
            rows_v.at[pl.ds(0, GSEQ)],
            out_hbm.at[pl.ds((base + GROUP * g) * SEQ, GSEQ)], sem)

    ib[pl.ds(96, 16)] = zeros16

    for c in range(NCHUNK):
        val = lane + jnp.full((16,), 16 * c + 1, jnp.int32)
        if c == NCHUNK - 1:
            val = jnp.where(tail_valid, val, zeros16)
        if c < 7:
            ia[pl.ds(16 * c, 16)] = val
        else:
            ib[pl.ds(16 * (c - 7), 16)] = val
    for s in range(GROUP):
        st_a, st_b = gather_cps(tab_v, SEQ * s, sg)
        st_a.start()
        st_b.start()
        st_a.wait()
        st_b.wait()

    def compute_pos(seq_ref, off):
        carry = zeros16
        for c in range(NCHUNK):
            v = seq_ref[pl.ds(off + 16 * c, 16)]
            nz = v != zeros16
            if c == NCHUNK - 1:
                nz = jnp.logical_and(nz, tail_valid)
            m = jnp.where(nz, ones16, zeros16)
            s = _cumsum16(m)
            pos = (s + carry) * m
            if c < 7:
                ia[pl.ds(16 * c, 16)] = pos
            else:
                ib[pl.ds(16 * (c - 7), 16)] = pos
            carry = carry + _lanegather(s, idx_last)

    def count_nonpad(seq_ref):
        acc = zeros16
        for c in range(GCHUNK):
            v = seq_ref[pl.ds(16 * c, 16)]
            acc = acc + jnp.where(v != zeros16, ones16, zeros16)
        return _allsum16(acc)[0]

    bufs = ((seq_g0, sin0, sout0),
            (seq_g1, sin1, sout1))

    flags[0] = 0
    flags[1] = 0

    in_cp(0, seq_g0, sin0).start()
    in_cp(1, seq_g1, sin1).start()

    def pair_loop(g2, carry_unused):
        for p in (0, 1):
            g = 2 * g2 + p
            seq_ref, sin, sout = bufs[p]
            in_cp(g, seq_ref, sin).wait()
            total = count_nonpad(seq_ref)

            @pl.when(flags[p] == 1)
            def _():
                out_fast_cp(g - 2, sout).wait()

            @pl.when(total == GSEQ)
            def _():
                out_fast_cp(g, sout).start()
                flags[p] = 1

            @pl.when(total != GSEQ)
            def _():
                for i in range(GROUP):
                    compute_pos(seq_ref, SEQ * i)
                    ga, gb = gather_cps(rows_v, SEQ * i, sg)
                    ga.start()
                    gb.start()
                    ga.wait()
                    gb.wait()
                cp = out_slow_cp(g, sout)
                cp.start()
                cp.wait()
                flags[p] = 0

            @pl.when(g2 < NGROUPS // 2 - 1)
            def _():
                in_cp(g + 2, seq_ref, sin).start()

        return carry_unused

    lax.fori_loop(0, NGROUPS // 2, pair_loop, jnp.int32(0))

    @pl.when(flags[0] == 1)
    def _():
        out_fast_cp(NGROUPS - 2, sout0).wait()

    @pl.when(flags[1] == 1)
    def _():
        out_fast_cp(NGROUPS - 1, sout1).wait()


@jax.jit
def _sc_call(seq, table):
    fn = functools.partial(
        pl.kernel,
        mesh=plsc.VectorSubcoreMesh(core_axis_name="c", subcore_axis_name="s"),
        compiler_params=pltpu.CompilerParams(use_tc_tiling_on_sc=False),
        out_type=jax.ShapeDtypeStruct((BATCH * SEQ, EMB), jnp.float32),
        scratch_types=[
            pltpu.VMEM((GSEQ + 8,), jnp.int32),
            pltpu.VMEM((GSEQ + 8,), jnp.int32),
            pltpu.VMEM((GA,), jnp.int32),
            pltpu.VMEM((GB,), jnp.int32),
            pltpu.VMEM((TAB_ROWS, EMB), jnp.float32),
            pltpu.VMEM((TAB_ROWS, EMB), jnp.float32),
            pltpu.SMEM((2,), jnp.int32),
            pltpu.SemaphoreType.DMA,
            pltpu.SemaphoreType.DMA,
            pltpu.SemaphoreType.DMA,
            pltpu.SemaphoreType.DMA,
            pltpu.SemaphoreType.DMA,
        ],
    )(_sc_body)
    return fn(seq, table)


def kernel(word_seq, position_enc_weight):
    seq = word_seq.astype(jnp.int32).reshape(-1)
    out = _sc_call(seq, position_enc_weight)
    return out.reshape(BATCH, SEQ, EMB)


# speedup vs baseline: 1.1218x; 1.1218x over previous
import functools

import jax
import jax.numpy as jnp
from jax import lax
from jax.experimental import pallas as pl
from jax.experimental.pallas import tpu as pltpu
from jax.experimental.pallas import tpu_sc as plsc

EMB = 64
SEQ = 200
SEQ_PAD = 208
NCHUNK = 13
BATCH = 4096
NWORKERS = 32
ROWS_PER_W = BATCH // NWORKERS
GROUP = 4
NGROUPS = ROWS_PER_W // GROUP
GSEQ = GROUP * SEQ
GCHUNK = GSEQ // 16
GA = 112
GB = 112
TAB_ROWS = GROUP * SEQ + 24


def _sc_body(seq_hbm, table_hbm, out_hbm,
             seq_g0, seq_g1, ia, ib, rows_v, tab_v, spm_tab, flags,
             sin0, sin1, sg, sout0, sout1):
    cid = lax.axis_index("c")
    sid = lax.axis_index("s")
    wid = sid * 2 + cid
    base = wid * ROWS_PER_W

    zeros16 = jnp.zeros((16,), jnp.int32)
    ones16 = jnp.ones((16,), jnp.int32)
    lane = lax.iota(jnp.int32, 16)
    tail_valid = lane < jnp.full((16,), 8, jnp.int32)
    scan_idx = [jnp.maximum(lane - (1 << k), zeros16) for k in range(4)]
    scan_msk = [lane >= jnp.full((16,), 1 << k, jnp.int32) for k in range(4)]
    bfly_idx = [lane ^ jnp.full((16,), 1 << k, jnp.int32) for k in range(4)]
    idx_last = jnp.full((16,), 15, jnp.int32)

    dnums = lax.GatherDimensionNumbers(
        offset_dims=(), collapsed_slice_dims=(0,), start_index_map=(0,))

    def _lanegather(x, idx):
        return lax.gather(x, idx[:, None], dnums, slice_sizes=(1,),
                          mode=lax.GatherScatterMode.PROMISE_IN_BOUNDS)

    def _cumsum16(m):
        s = m
        for k in range(4):
            g = _lanegather(s, scan_idx[k])
            s = s + jnp.where(scan_msk[k], g, zeros16)
        return s

    def _allsum16(x):
        s = x
        for k in range(4):
            s = s + _lanegather(s, bfly_idx[k])
        return s

    def gather_cps(rows_ref, row_off, sem):
        return (pltpu.make_async_copy(
                    table_hbm.at[ia],
                    rows_ref.at[pl.ds(row_off, GA)], sem),
                pltpu.make_async_copy(
                    table_hbm.at[ib],
                    rows_ref.at[pl.ds(row_off + GA, GB)], sem))

    def in_cp(g, seq_ref, sem):
        return pltpu.make_async_copy(
            seq_hbm.at[pl.ds((base + GROUP * g) * SEQ, GSEQ)],
            seq_ref.at[pl.ds(0, GSEQ)], sem)

    def out_fast_cp(g, sem):
        return pltpu.make_async_copy(
            spm_tab.at[pl.ds(0, GSEQ)],
            out_hbm.at[pl.ds((base + GROUP * g) * SEQ, GSEQ)], sem)

    def out_slow_cp(g, sem):
        return pltpu.make_async_copy(
            rows_v.at[pl.ds(0, GSEQ)],
            out_hbm.at[pl.ds((base + GROUP * g) * SEQ, GSEQ)], sem)

    ib[pl.ds(96, 16)] = zeros16

    for c in range(NCHUNK):
        val = lane + jnp.full((16,), 16 * c + 1, jnp.int32)
        if c == NCHUNK - 1:
            val = jnp.where(tail_valid, val, zeros16)
        if c < 7:
            ia[pl.ds(16 * c, 16)] = val
        else:
            ib[pl.ds(16 * (c - 7), 16)] = val

    @pl.when(sid == 0)
    def _():
        for s in range(GROUP):
            st_a, st_b = gather_cps(tab_v, SEQ * s, sg)
            st_a.start()
            st_b.start()
            st_a.wait()
            st_b.wait()
        pltpu.sync_copy(tab_v.at[pl.ds(0, GSEQ)], spm_tab.at[pl.ds(0, GSEQ)])

    plsc.subcore_barrier()

    def compute_pos(seq_ref, off):
        carry = zeros16
        for c in range(NCHUNK):
            v = seq_ref[pl.ds(off + 16 * c, 16)]
            nz = v != zeros16
            if c == NCHUNK - 1:
                nz = jnp.logical_and(nz, tail_valid)
            m = jnp.where(nz, ones16, zeros16)
            s = _cumsum16(m)
            pos = (s + carry) * m
            if c < 7:
                ia[pl.ds(16 * c, 16)] = pos
            else:
                ib[pl.ds(16 * (c - 7), 16)] = pos
            carry = carry + _lanegather(s, idx_last)

    def count_nonpad(seq_ref):
        acc = zeros16
        for c in range(GCHUNK):
            v = seq_ref[pl.ds(16 * c, 16)]
            acc = acc + jnp.where(v != zeros16, ones16, zeros16)
        return _allsum16(acc)[0]

    bufs = ((seq_g0, sin0, sout0),
            (seq_g1, sin1, sout1))

    flags[0] = 0
    flags[1] = 0

    in_cp(0, seq_g0, sin0).start()
    in_cp(1, seq_g1, sin1).start()

    def pair_loop(g2, carry_unused):
        for p in (0, 1):
            g = 2 * g2 + p
            seq_ref, sin, sout = bufs[p]
            in_cp(g, seq_ref, sin).wait()
            total = count_nonpad(seq_ref)

            @pl.when(flags[p] == 1)
            def _():
                out_fast_cp(g - 2, sout).wait()

            @pl.when(total == GSEQ)
            def _():
                out_fast_cp(g, sout).start()
                flags[p] = 1

            @pl.when(total != GSEQ)
            def _():
                for i in range(GROUP):
                    compute_pos(seq_ref, SEQ * i)
                    ga, gb = gather_cps(rows_v, SEQ * i, sg)
                    ga.start()
                    gb.start()
                    ga.wait()
                    gb.wait()
                cp = out_slow_cp(g, sout)
                cp.start()
                cp.wait()
                flags[p] = 0

            @pl.when(g2 < NGROUPS // 2 - 1)
            def _():
                in_cp(g + 2, seq_ref, sin).start()

        return carry_unused

    lax.fori_loop(0, NGROUPS // 2, pair_loop, jnp.int32(0))

    @pl.when(flags[0] == 1)
    def _():
        out_fast_cp(NGROUPS - 2, sout0).wait()

    @pl.when(flags[1] == 1)
    def _():
        out_fast_cp(NGROUPS - 1, sout1).wait()


@jax.jit
def _sc_call(seq, table):
    fn = functools.partial(
        pl.kernel,
        mesh=plsc.VectorSubcoreMesh(core_axis_name="c", subcore_axis_name="s"),
        compiler_params=pltpu.CompilerParams(use_tc_tiling_on_sc=False),
        out_type=jax.ShapeDtypeStruct((BATCH * SEQ, EMB), jnp.float32),
        scratch_types=[
            pltpu.VMEM((GSEQ + 8,), jnp.int32),
            pltpu.VMEM((GSEQ + 8,), jnp.int32),
            pltpu.VMEM((GA,), jnp.int32),
            pltpu.VMEM((GB,), jnp.int32),
            pltpu.VMEM((TAB_ROWS, EMB), jnp.float32),
            pltpu.VMEM((TAB_ROWS, EMB), jnp.float32),
            pltpu.VMEM_SHARED((GSEQ, EMB), jnp.float32),
            pltpu.SMEM((2,), jnp.int32),
            pltpu.SemaphoreType.DMA,
            pltpu.SemaphoreType.DMA,
            pltpu.SemaphoreType.DMA,
            pltpu.SemaphoreType.DMA,
            pltpu.SemaphoreType.DMA,
        ],
    )(_sc_body)
    return fn(seq, table)


def kernel(word_seq, position_enc_weight):
    seq = word_seq.astype(jnp.int32).reshape(-1)
    out = _sc_call(seq, position_enc_weight)
    return out.reshape(BATCH, SEQ, EMB)


# speedup vs baseline: 1.1914x; 1.0621x over previous
import functools

import jax
import jax.numpy as jnp
from jax import lax
from jax.experimental import pallas as pl
from jax.experimental.pallas import tpu as pltpu
from jax.experimental.pallas import tpu_sc as plsc

EMB = 64
SEQ = 200
NCHUNK = 13
BATCH = 4096
NWORKERS = 32
ROWS_PER_W = BATCH // NWORKERS
GROUP = 4
NGROUPS = ROWS_PER_W // GROUP
GSEQ = GROUP * SEQ
GCHUNK = GSEQ // 16
GA = 112
GB = 88


def _sc_body(seq_hbm, table_hbm, out_hbm,
             seq_g0, seq_g1, ia, ib, rows_v, tab_v, spm_tab, flags,
             sin0, sin1, sg, sout0, sout1):
    cid = lax.axis_index("c")
    sid = lax.axis_index("s")
    wid = sid * 2 + cid
    base = wid * ROWS_PER_W

    zeros16 = jnp.zeros((16,), jnp.int32)
    ones16 = jnp.ones((16,), jnp.int32)
    lane = lax.iota(jnp.int32, 16)
    tail_valid = lane < jnp.full((16,), 8, jnp.int32)
    scan_idx = [jnp.maximum(lane - (1 << k), zeros16) for k in range(4)]
    scan_msk = [lane >= jnp.full((16,), 1 << k, jnp.int32) for k in range(4)]
    bfly_idx = [lane ^ jnp.full((16,), 1 << k, jnp.int32) for k in range(4)]
    idx_last = jnp.full((16,), 15, jnp.int32)

    dnums = lax.GatherDimensionNumbers(
        offset_dims=(), collapsed_slice_dims=(0,), start_index_map=(0,))

    def _lanegather(x, idx):
        return lax.gather(x, idx[:, None], dnums, slice_sizes=(1,),
                          mode=lax.GatherScatterMode.PROMISE_IN_BOUNDS)

    def _cumsum16(m):
        s = m
        for k in range(4):
            g = _lanegather(s, scan_idx[k])
            s = s + jnp.where(scan_msk[k], g, zeros16)
        return s

    def _allsum16(x):
        s = x
        for k in range(4):
            s = s + _lanegather(s, bfly_idx[k])
        return s

    def gather_cps(dst3, slot, sem):
        return (pltpu.make_async_copy(
                    table_hbm.at[ia],
                    dst3.at[slot, pl.ds(0, GA)], sem),
                pltpu.make_async_copy(
                    table_hbm.at[ib.at[pl.ds(0, GB)]],
                    dst3.at[slot, pl.ds(GA, GB)], sem))

    def in_cp(g, seq_ref, sem):
        return pltpu.make_async_copy(
            seq_hbm.at[pl.ds((base + GROUP * g) * SEQ, GSEQ)],
            seq_ref.at[pl.ds(0, GSEQ)], sem)

    def out_fast_cp(g, sem):
        return pltpu.make_async_copy(
            spm_tab, out_hbm.at[pl.ds(base + GROUP * g, GROUP)], sem)

    def out_slow_cp(g, sem):
        return pltpu.make_async_copy(
            rows_v, out_hbm.at[pl.ds(base + GROUP * g, GROUP)], sem)

    for c in range(NCHUNK):
        val = lane + jnp.full((16,), 16 * c + 1, jnp.int32)
        if c == NCHUNK - 1:
            val = jnp.where(tail_valid, val, zeros16)
        if c < 7:
            ia[pl.ds(16 * c, 16)] = val
        else:
            ib[pl.ds(16 * (c - 7), 16)] = val

    @pl.when(sid == 0)
    def _():
        for s in range(GROUP):
            st_a, st_b = gather_cps(tab_v, s, sg)
            st_a.start()
            st_b.start()
            st_a.wait()
            st_b.wait()
        pltpu.sync_copy(tab_v, spm_tab)

    plsc.subcore_barrier()

    def compute_pos(seq_ref, off):
        carry = zeros16
        for c in range(NCHUNK):
            v = seq_ref[pl.ds(off + 16 * c, 16)]
            nz = v != zeros16
            if c == NCHUNK - 1:
                nz = jnp.logical_and(nz, tail_valid)
            m = jnp.where(nz, ones16, zeros16)
            s = _cumsum16(m)
            pos = (s + carry) * m
            if c < 7:
                ia[pl.ds(16 * c, 16)] = pos
            else:
                ib[pl.ds(16 * (c - 7), 16)] = pos
            carry = carry + _lanegather(s, idx_last)

    def count_nonpad(seq_ref):
        acc = zeros16
        for c in range(GCHUNK):
            v = seq_ref[pl.ds(16 * c, 16)]
            acc = acc + jnp.where(v != zeros16, ones16, zeros16)
        return _allsum16(acc)[0]

    bufs = ((seq_g0, sin0, sout0),
            (seq_g1, sin1, sout1))

    flags[0] = 0
    flags[1] = 0

    in_cp(0, seq_g0, sin0).start()
    in_cp(1, seq_g1, sin1).start()

    def pair_loop(g2, carry_unused):
        for p in (0, 1):
            g = 2 * g2 + p
            seq_ref, sin, sout = bufs[p]
            in_cp(g, seq_ref, sin).wait()
            total = count_nonpad(seq_ref)

            @pl.when(flags[p] == 1)
            def _():
                out_fast_cp(g - 2, sout).wait()

            @pl.when(total == GSEQ)
            def _():
                out_fast_cp(g, sout).start()
                flags[p] = 1

            @pl.when(total != GSEQ)
            def _():
                for i in range(GROUP):
                    compute_pos(seq_ref, SEQ * i)
                    ga, gb = gather_cps(rows_v, i, sg)
                    ga.start()
                    gb.start()
                    ga.wait()
                    gb.wait()
                cp = out_slow_cp(g, sout)
                cp.start()
                cp.wait()
                flags[p] = 0

            @pl.when(g2 < NGROUPS // 2 - 1)
            def _():
                in_cp(g + 2, seq_ref, sin).start()

        return carry_unused

    lax.fori_loop(0, NGROUPS // 2, pair_loop, jnp.int32(0))

    @pl.when(flags[0] == 1)
    def _():
        out_fast_cp(NGROUPS - 2, sout0).wait()

    @pl.when(flags[1] == 1)
    def _():
        out_fast_cp(NGROUPS - 1, sout1).wait()


@jax.jit
def _sc_call(seq, table):
    fn = functools.partial(
        pl.kernel,
        mesh=plsc.VectorSubcoreMesh(core_axis_name="c", subcore_axis_name="s"),
        compiler_params=pltpu.CompilerParams(use_tc_tiling_on_sc=False),
        out_type=jax.ShapeDtypeStruct((BATCH, SEQ, EMB), jnp.float32),
        scratch_types=[
            pltpu.VMEM((GSEQ + 8,), jnp.int32),
            pltpu.VMEM((GSEQ + 8,), jnp.int32),
            pltpu.VMEM((GA,), jnp.int32),
            pltpu.VMEM((96,), jnp.int32),
            pltpu.VMEM((GROUP, SEQ, EMB), jnp.float32),
            pltpu.VMEM((GROUP, SEQ, EMB), jnp.float32),
            pltpu.VMEM_SHARED((GROUP, SEQ, EMB), jnp.float32),
            pltpu.SMEM((2,), jnp.int32),
            pltpu.SemaphoreType.DMA,
            pltpu.SemaphoreType.DMA,
            pltpu.SemaphoreType.DMA,
            pltpu.SemaphoreType.DMA,
            pltpu.SemaphoreType.DMA,
        ],
    )(_sc_body)
    return fn(seq, table)


def kernel(word_seq, position_enc_weight):
    seq = word_seq.astype(jnp.int32).reshape(-1)
    return _sc_call(seq, position_enc_weight)
